# baseline (speedup 1.0000x reference)
import functools

import jax
import jax.numpy as jnp
from jax import lax
from jax.experimental import pallas as pl
from jax.experimental.pallas import tpu as pltpu
from jax.experimental.pallas import tpu_sc as plsc

_B, _S = 1024, 200
_D = 32
_N = _B * _S
_CHUNK = 128
_NBUF = 5


@functools.cache
def _build_gather():
    info = plsc.get_sparse_core_info()
    nw = info.num_cores * info.num_subcores
    n_per_w = _N // nw
    nchunks = n_per_w // _CHUNK
    assert nchunks % _NBUF == 0

    mesh = plsc.VectorSubcoreMesh(core_axis_name="c", subcore_axis_name="s")

    @functools.partial(
        pl.kernel,
        out_type=jax.ShapeDtypeStruct((_N, _D), jnp.float32),
        mesh=mesh,
        scratch_types=[
            pltpu.VMEM((nchunks, _CHUNK), jnp.int32),
            pltpu.VMEM((_NBUF, _CHUNK, _D), jnp.float32),
            pltpu.SemaphoreType.DMA((_NBUF,)),
        ],
        compiler_params=pltpu.CompilerParams(use_tc_tiling_on_sc=False),
    )
    def gather(idx_hbm, table_hbm, out_hbm, idx_v, bufs, sems):
        wid = lax.axis_index("s") * info.num_cores + lax.axis_index("c")
        base = wid * n_per_w

        pltpu.sync_copy(idx_hbm.at[wid], idx_v)

        for b in range(_NBUF):
            pltpu.async_copy(table_hbm.at[idx_v.at[b]], bufs.at[b], sems.at[b])

        def outer(i, carry):
            o = i * _NBUF
            for b in range(_NBUF):
                c = o + b
                pltpu.make_async_copy(
                    table_hbm.at[idx_v.at[0]], bufs.at[b], sems.at[b]
                ).wait()
                pltpu.sync_copy(
                    bufs.at[b], out_hbm.at[pl.ds(base + c * _CHUNK, _CHUNK)]
                )
                nxt = c + _NBUF

                @pl.when(nxt < nchunks)
                def _fire():
                    pltpu.async_copy(
                        table_hbm.at[idx_v.at[nxt]], bufs.at[b], sems.at[b]
                    )

            return carry

        lax.fori_loop(0, nchunks // _NBUF, outer, None)

    return gather, nw, nchunks


def kernel(token_ids, weight):
    gather, nw, nchunks = _build_gather()
    idx = token_ids.reshape(nw, nchunks, _CHUNK).astype(jnp.int32)
    out = gather(idx, weight)
    return out.reshape(_B, _S, _D)


# speedup vs baseline: 1.0042x; 1.0042x over previous
import functools

import jax
import jax.numpy as jnp
from jax import lax
from jax.experimental import pallas as pl
from jax.experimental.pallas import tpu as pltpu
from jax.experimental.pallas import tpu_sc as plsc

_B, _S = 1024, 200
_D = 32
_N = _B * _S
_CHUNK = 640
_NBUF = 5


@functools.cache
def _build_gather():
    info = plsc.get_sparse_core_info()
    nw = info.num_cores * info.num_subcores
    n_per_w = _N // nw
    nchunks = n_per_w // _CHUNK
    assert nchunks % _NBUF == 0

    mesh = plsc.VectorSubcoreMesh(core_axis_name="c", subcore_axis_name="s")

    @functools.partial(
        pl.kernel,
        out_type=jax.ShapeDtypeStruct((_N, _D), jnp.float32),
        mesh=mesh,
        scratch_types=[
            pltpu.VMEM((nchunks, _CHUNK), jnp.int32),
            pltpu.VMEM((_NBUF, _CHUNK, _D), jnp.float32),
            pltpu.SemaphoreType.DMA((_NBUF,)),
        ],
        compiler_params=pltpu.CompilerParams(use_tc_tiling_on_sc=False),
    )
    def gather(idx_hbm, table_hbm, out_hbm, idx_v, bufs, sems):
        wid = lax.axis_index("s") * info.num_cores + lax.axis_index("c")
        base = wid * n_per_w

        pltpu.sync_copy(idx_hbm.at[wid], idx_v)

        for b in range(_NBUF):
            pltpu.async_copy(table_hbm.at[idx_v.at[b]], bufs.at[b], sems.at[b])

        def outer(i, carry):
            o = i * _NBUF
            for b in range(_NBUF):
                c = o + b
                pltpu.make_async_copy(
                    table_hbm.at[idx_v.at[0]], bufs.at[b], sems.at[b]
                ).wait()
                pltpu.sync_copy(
                    bufs.at[b], out_hbm.at[pl.ds(base + c * _CHUNK, _CHUNK)]
                )
                nxt = c + _NBUF

                @pl.when(nxt < nchunks)
                def _fire():
                    pltpu.async_copy(
                        table_hbm.at[idx_v.at[nxt]], bufs.at[b], sems.at[b]
                    )

            return carry

        lax.fori_loop(0, nchunks // _NBUF, outer, None)

    return gather, nw, nchunks


def kernel(token_ids, weight):
    gather, nw, nchunks = _build_gather()
    idx = token_ids.reshape(nw, nchunks, _CHUNK).astype(jnp.int32)
    out = gather(idx, weight)
    return out.reshape(_B, _S, _D)
